# baseline (speedup 1.0000x reference)
import functools

import jax
import jax.numpy as jnp
from jax import lax
from jax.experimental import pallas as pl
from jax.experimental.pallas import tpu as pltpu
from jax.experimental.pallas import tpu_sc as plsc

B = 4096
D = 1024
CD = 2 * D
H = 1024
EH = 512
E = 8
K = 2

GATE_BT = 512
EXP_BT = 1024

_NC, _NS, _L = 2, 16, 16
_NW = _NC * _NS
_TOK_W = B // _NW
_CH = _TOK_W * E


def _silu(x):
    return x * jax.nn.sigmoid(x)


def _silu_fast(x):
    xh = 0.5 * x
    return xh + xh * jnp.tanh(xh)


def _gates_body(x1_ref, x2_ref, gaW1_ref, gab1_ref, gaW2_ref, gab2_ref,
                gbW1_ref, gbb1_ref, gbW2_ref, gbb2_ref,
                eW1_ref, eW2_ref, aW1_ref, bW1_ref,
                sa_ref, sb_ref, saT_ref, sbT_ref, xbf_ref,
                eW1b_ref, eW2b_ref, aW1b_ref, bW1b_ref):
    i = pl.program_id(0)
    x = jnp.concatenate([x1_ref[...], x2_ref[...]], axis=1)
    xbf_ref[...] = x.astype(jnp.bfloat16)
    eW1b_ref[...] = eW1_ref[...].astype(jnp.bfloat16)
    eW2b_ref[...] = eW2_ref[...].astype(jnp.bfloat16)

    @pl.when(i == 0)
    def _():
        aW1b_ref[...] = aW1_ref[...].astype(jnp.bfloat16)
        bW1b_ref[...] = bW1_ref[...].astype(jnp.bfloat16)

    for w1_ref, b1_ref, w2_ref, b2_ref, out_ref, outT_ref in (
            (gaW1_ref, gab1_ref, gaW2_ref, gab2_ref, sa_ref, saT_ref),
            (gbW1_ref, gbb1_ref, gbW2_ref, gbb2_ref, sb_ref, sbT_ref)):
        h = _silu_fast(jnp.dot(x, w1_ref[...],
                               preferred_element_type=jnp.float32)
                       + b1_ref[...])
        logits = jnp.dot(h, w2_ref[...],
                         preferred_element_type=jnp.float32) + b2_ref[...]
        m = jnp.max(logits, axis=1, keepdims=True)
        ex = jnp.exp(logits - m)
        scores = ex / jnp.sum(ex, axis=1, keepdims=True)
        out_ref[...] = scores
        outT_ref[...] = jnp.transpose(scores, (1, 0))


def _gates_call(x1, x2, gaW1, gab1, gaW2, gab2, gbW1, gbb1, gbW2, gbb2,
                eW1, eW2, aW1, bW1):
    full = lambda shape: pl.BlockSpec(shape, lambda i: (0,) * len(shape))
    bf = jnp.bfloat16
    return pl.pallas_call(
        _gates_body,
        grid=(B // GATE_BT,),
        in_specs=[
            pl.BlockSpec((GATE_BT, D), lambda i: (i, 0)),
            pl.BlockSpec((GATE_BT, D), lambda i: (i, 0)),
            full((CD, H)), full((1, H)), full((H, E)), full((1, E)),
            full((CD, H)), full((1, H)), full((H, E)), full((1, E)),
            pl.BlockSpec((1, CD, EH), lambda i: (i, 0, 0)),
            pl.BlockSpec((1, EH, EH), lambda i: (i, 0, 0)),
            full((EH, EH)), full((EH, EH)),
        ],
        out_specs=[
            pl.BlockSpec((GATE_BT, E), lambda i: (i, 0)),
            pl.BlockSpec((GATE_BT, E), lambda i: (i, 0)),
            pl.BlockSpec((E, GATE_BT), lambda i: (0, i)),
            pl.BlockSpec((E, GATE_BT), lambda i: (0, i)),
            pl.BlockSpec((GATE_BT, CD), lambda i: (i, 0)),
            pl.BlockSpec((1, CD, EH), lambda i: (i, 0, 0)),
            pl.BlockSpec((1, EH, EH), lambda i: (i, 0, 0)),
            full((EH, EH)), full((EH, EH)),
        ],
        out_shape=[
            jax.ShapeDtypeStruct((B, E), jnp.float32),
            jax.ShapeDtypeStruct((B, E), jnp.float32),
            jax.ShapeDtypeStruct((E, B), jnp.float32),
            jax.ShapeDtypeStruct((E, B), jnp.float32),
            jax.ShapeDtypeStruct((B, CD), bf),
            jax.ShapeDtypeStruct((E, CD, EH), bf),
            jax.ShapeDtypeStruct((E, EH, EH), bf),
            jax.ShapeDtypeStruct((EH, EH), bf),
            jax.ShapeDtypeStruct((EH, EH), bf),
        ],
        compiler_params=pltpu.CompilerParams(
            dimension_semantics=("arbitrary",)),
    )(x1, x2, gaW1, gab1, gaW2, gab2, gbW1, gbb1, gbW2, gbb2,
      eW1, eW2, aW1, bW1)


def _route_body(sa_hbm, sb_hbm, wa_hbm, wb_hbm, sa_v, sb_v, wa_v, wb_v):
    wid = lax.axis_index("s") * _NC + lax.axis_index("c")
    base = wid * _TOK_W
    pltpu.sync_copy(sa_hbm.at[:, pl.ds(base, _TOK_W)], sa_v)
    pltpu.sync_copy(sb_hbm.at[:, pl.ds(base, _TOK_W)], sb_v)
    for g in range(_TOK_W // _L):
        for s_v, w_v in ((sa_v, wa_v), (sb_v, wb_v)):
            best1 = jnp.full((_L,), -1.0, jnp.float32)
            best2 = jnp.full((_L,), -1.0, jnp.float32)
            idx1 = jnp.zeros((_L,), jnp.int32)
            idx2 = jnp.zeros((_L,), jnp.int32)
            for e in range(E):
                v = s_v[e, pl.ds(g * _L, _L)]
                gt1 = v > best1
                gt2 = v > best2
                best2 = jnp.where(gt1, best1, jnp.where(gt2, v, best2))
                idx2 = jnp.where(gt1, idx1, jnp.where(gt2, e, idx2))
                best1 = jnp.where(gt1, v, best1)
                idx1 = jnp.where(gt1, e, idx1)
            for e in range(E):
                w_v[e, pl.ds(g * _L, _L)] = (
                    jnp.where(idx1 == e, best1, 0.0)
                    + jnp.where(idx2 == e, best2, 0.0))
    pltpu.sync_copy(wa_v, wa_hbm.at[:, pl.ds(base, _TOK_W)])
    pltpu.sync_copy(wb_v, wb_hbm.at[:, pl.ds(base, _TOK_W)])


def _route_call(scores_aT, scores_bT):
    mesh = plsc.VectorSubcoreMesh(core_axis_name="c", subcore_axis_name="s",
                                  num_cores=_NC, num_subcores=_NS)
    f = pl.kernel(
        _route_body,
        out_type=(jax.ShapeDtypeStruct((E, B), jnp.float32),
                  jax.ShapeDtypeStruct((E, B), jnp.float32)),
        mesh=mesh,
        scratch_types=[pltpu.VMEM((E, _TOK_W), jnp.float32),
                       pltpu.VMEM((E, _TOK_W), jnp.float32),
                       pltpu.VMEM((E, _TOK_W), jnp.float32),
                       pltpu.VMEM((E, _TOK_W), jnp.float32)],
        compiler_params=pltpu.CompilerParams(needs_layout_passes=False),
    )
    return f(scores_aT, scores_bT)


def _experts_body(x_ref, w1_ref, b1_ref, w2_ref, b2_ref, wa_ref, wb_ref,
                  agg_a_ref, agg_b_ref):
    x = x_ref[...]
    wT = jnp.transpose(wa_ref[...], (1, 0)).astype(jnp.bfloat16)
    wbTl = jnp.transpose(wb_ref[...], (1, 0)).astype(jnp.bfloat16)
    agg_a = None
    for e in range(E):
        h1 = _silu_fast(jnp.dot(x, w1_ref[e],
                                preferred_element_type=jnp.float32)
                        .astype(jnp.bfloat16) + b1_ref[e])
        h2 = _silu_fast(jnp.dot(h1, w2_ref[e],
                                preferred_element_type=jnp.float32)
                        .astype(jnp.bfloat16) + b2_ref[e])
        wa_col = wT[:, e:e + 1]
        wb_col = wbTl[:, e:e + 1]
        if agg_a is None:
            agg_a = wa_col * h2
            agg_b = wb_col * h2
        else:
            agg_a += wa_col * h2
            agg_b += wb_col * h2
    agg_a_ref[...] = agg_a
    agg_b_ref[...] = agg_b


def _heads_body(agg_a_ref, agg_b_ref,
                aW1_ref, ab1_ref, aW2_ref, ab2_ref,
                bW1_ref, bb1_ref, bW2_ref, bb2_ref,
                alpha_ref, beta_ref):
    for agg_ref, hW1_ref, hb1_ref, hW2_ref, hb2_ref, out_ref in (
            (agg_a_ref, aW1_ref, ab1_ref, aW2_ref, ab2_ref, alpha_ref),
            (agg_b_ref, bW1_ref, bb1_ref, bW2_ref, bb2_ref, beta_ref)):
        hh = _silu_fast(jnp.dot(agg_ref[...], hW1_ref[...],
                                preferred_element_type=jnp.float32)
                        + hb1_ref[...])
        logit = (jnp.sum(hh * hW2_ref[...], axis=1, keepdims=True)
                 + hb2_ref[...])
        out_ref[...] = jax.nn.softplus(logit)


def _heads_call(agg_a, agg_b, aW1b, ab1r, aW2r, ab2r, bW1b, bb1r, bW2r, bb2r):
    full = lambda shape: pl.BlockSpec(shape, lambda i: (0,) * len(shape))
    return pl.pallas_call(
        _heads_body,
        grid=(B // EXP_BT,),
        in_specs=[
            pl.BlockSpec((EXP_BT, EH), lambda i: (i, 0)),
            pl.BlockSpec((EXP_BT, EH), lambda i: (i, 0)),
            full((EH, EH)), full((1, EH)), full((1, EH)), full((1, 1)),
            full((EH, EH)), full((1, EH)), full((1, EH)), full((1, 1)),
        ],
        out_specs=[
            pl.BlockSpec((EXP_BT, 1), lambda i: (i, 0)),
            pl.BlockSpec((EXP_BT, 1), lambda i: (i, 0)),
        ],
        out_shape=[
            jax.ShapeDtypeStruct((B, 1), jnp.float32),
            jax.ShapeDtypeStruct((B, 1), jnp.float32),
        ],
        compiler_params=pltpu.CompilerParams(
            dimension_semantics=("arbitrary",)),
    )(agg_a, agg_b, aW1b, ab1r, aW2r, ab2r, bW1b, bb1r, bW2r, bb2r)


def _experts_call(xb, eW1b, eb1r, eW2b, eb2r, wa, wb):
    full = lambda shape: pl.BlockSpec(shape, lambda i: (0,) * len(shape))
    return pl.pallas_call(
        _experts_body,
        grid=(B // EXP_BT,),
        in_specs=[
            pl.BlockSpec((EXP_BT, CD), lambda i: (i, 0)),
            full((E, CD, EH)), full((E, 1, EH)),
            full((E, EH, EH)), full((E, 1, EH)),
            pl.BlockSpec((E, EXP_BT), lambda i: (0, i)),
            pl.BlockSpec((E, EXP_BT), lambda i: (0, i)),
        ],
        out_specs=[
            pl.BlockSpec((EXP_BT, EH), lambda i: (i, 0)),
            pl.BlockSpec((EXP_BT, EH), lambda i: (i, 0)),
        ],
        out_shape=[
            jax.ShapeDtypeStruct((B, EH), jnp.bfloat16),
            jax.ShapeDtypeStruct((B, EH), jnp.bfloat16),
        ],
        compiler_params=pltpu.CompilerParams(
            dimension_semantics=("arbitrary",),
            vmem_limit_bytes=100 * 1024 * 1024),
    )(xb, eW1b, eb1r, eW2b, eb2r, wa, wb)


def kernel(inputs, reference_base_embedding,
           gaW1, gab1, gaW2, gab2,
           gbW1, gbb1, gbW2, gbb2,
           eW1, eb1, eW2, eb2,
           aW1, ab1, aW2, ab2,
           bW1, bb1, bW2, bb2):
    (scores_a, scores_b, saT, sbT, xbf, eW1b, eW2b, aW1b, bW1b) = _gates_call(
        inputs, reference_base_embedding,
        gaW1, gab1.reshape(1, H), gaW2, gab2.reshape(1, E),
        gbW1, gbb1.reshape(1, H), gbW2, gbb2.reshape(1, E),
        eW1, eW2, aW1, bW1)

    waT, wbT = _route_call(saT, sbT)

    bf = jnp.bfloat16
    agg_a, agg_b = _experts_call(
        xbf,
        eW1b, eb1.reshape(E, 1, EH).astype(bf),
        eW2b, eb2.reshape(E, 1, EH).astype(bf),
        waT, wbT)

    alpha, beta = _heads_call(
        agg_a, agg_b,
        aW1b, ab1.reshape(1, EH), aW2.reshape(1, EH), ab2.reshape(1, 1),
        bW1b, bb1.reshape(1, EH), bW2.reshape(1, EH), bb2.reshape(1, 1))

    return (alpha, beta, scores_a, scores_b)


# speedup vs baseline: 1.0049x; 1.0049x over previous
import jax
import jax.numpy as jnp
from jax import lax
from jax.experimental import pallas as pl
from jax.experimental.pallas import tpu as pltpu
from jax.experimental.pallas import tpu_sc as plsc

B = 4096
D = 1024
CD = 2 * D
H = 1024
EH = 512
E = 8

GATE_BT = 512
EXP_BT = 512

_NC, _NS, _L = 2, 16, 16
_NW = _NC * _NS
_TOK_W = B // _NW


def _silu_fast(x):
    xh = 0.5 * x
    return xh + xh * jnp.tanh(xh)


def _gates_body(x1_ref, x2_ref, gaW1_ref, gab1_ref, gaW2_ref, gab2_ref,
                gbW1_ref, gbb1_ref, gbW2_ref, gbb2_ref,
                eW1_ref, eW2_ref, aW1_ref, bW1_ref,
                sa_ref, sb_ref, saT_ref, sbT_ref, xbf_ref,
                eW1b_ref, eW2b_ref, aW1b_ref, bW1b_ref):
    i = pl.program_id(0)
    x = jnp.concatenate([x1_ref[...], x2_ref[...]], axis=1)
    xbf_ref[...] = x.astype(jnp.bfloat16)
    eW1b_ref[...] = eW1_ref[...].astype(jnp.bfloat16)
    eW2b_ref[...] = eW2_ref[...].astype(jnp.bfloat16)

    @pl.when(i == 0)
    def _():
        aW1b_ref[...] = aW1_ref[...].astype(jnp.bfloat16)
        bW1b_ref[...] = bW1_ref[...].astype(jnp.bfloat16)

    for w1_ref, b1_ref, w2_ref, b2_ref, out_ref, outT_ref in (
            (gaW1_ref, gab1_ref, gaW2_ref, gab2_ref, sa_ref, saT_ref),
            (gbW1_ref, gbb1_ref, gbW2_ref, gbb2_ref, sb_ref, sbT_ref)):
        h = _silu_fast(jnp.dot(x, w1_ref[...],
                               preferred_element_type=jnp.float32)
                       + b1_ref[...])
        logits = jnp.dot(h, w2_ref[...],
                         preferred_element_type=jnp.float32) + b2_ref[...]
        m = jnp.max(logits, axis=1, keepdims=True)
        ex = jnp.exp(logits - m)
        scores = ex / jnp.sum(ex, axis=1, keepdims=True)
        out_ref[...] = scores
        outT_ref[...] = jnp.transpose(scores, (1, 0))


def _gates_call(x1, x2, gaW1, gab1, gaW2, gab2, gbW1, gbb1, gbW2, gbb2,
                eW1, eW2, aW1, bW1):
    full = lambda shape: pl.BlockSpec(shape, lambda i: (0,) * len(shape))
    bf = jnp.bfloat16
    return pl.pallas_call(
        _gates_body,
        grid=(B // GATE_BT,),
        in_specs=[
            pl.BlockSpec((GATE_BT, D), lambda i: (i, 0)),
            pl.BlockSpec((GATE_BT, D), lambda i: (i, 0)),
            full((CD, H)), full((1, H)), full((H, E)), full((1, E)),
            full((CD, H)), full((1, H)), full((H, E)), full((1, E)),
            pl.BlockSpec((1, CD, EH), lambda i: (i, 0, 0)),
            pl.BlockSpec((1, EH, EH), lambda i: (i, 0, 0)),
            full((EH, EH)), full((EH, EH)),
        ],
        out_specs=[
            pl.BlockSpec((GATE_BT, E), lambda i: (i, 0)),
            pl.BlockSpec((GATE_BT, E), lambda i: (i, 0)),
            pl.BlockSpec((E, GATE_BT), lambda i: (0, i)),
            pl.BlockSpec((E, GATE_BT), lambda i: (0, i)),
            pl.BlockSpec((GATE_BT, CD), lambda i: (i, 0)),
            pl.BlockSpec((1, CD, EH), lambda i: (i, 0, 0)),
            pl.BlockSpec((1, EH, EH), lambda i: (i, 0, 0)),
            full((EH, EH)), full((EH, EH)),
        ],
        out_shape=[
            jax.ShapeDtypeStruct((B, E), jnp.float32),
            jax.ShapeDtypeStruct((B, E), jnp.float32),
            jax.ShapeDtypeStruct((E, B), jnp.float32),
            jax.ShapeDtypeStruct((E, B), jnp.float32),
            jax.ShapeDtypeStruct((B, CD), bf),
            jax.ShapeDtypeStruct((E, CD, EH), bf),
            jax.ShapeDtypeStruct((E, EH, EH), bf),
            jax.ShapeDtypeStruct((EH, EH), bf),
            jax.ShapeDtypeStruct((EH, EH), bf),
        ],
        compiler_params=pltpu.CompilerParams(
            dimension_semantics=("arbitrary",)),
    )(x1, x2, gaW1, gab1, gaW2, gab2, gbW1, gbb1, gbW2, gbb2,
      eW1, eW2, aW1, bW1)


def _route_body(sa_hbm, sb_hbm, wa_hbm, wb_hbm, sa_v, sb_v, wa_v, wb_v):
    wid = lax.axis_index("s") * _NC + lax.axis_index("c")
    base = wid * _TOK_W
    pltpu.sync_copy(sa_hbm.at[:, pl.ds(base, _TOK_W)], sa_v)
    pltpu.sync_copy(sb_hbm.at[:, pl.ds(base, _TOK_W)], sb_v)
    for g in range(_TOK_W // _L):
        for s_v, w_v in ((sa_v, wa_v), (sb_v, wb_v)):
            best1 = jnp.full((_L,), -1.0, jnp.float32)
            best2 = jnp.full((_L,), -1.0, jnp.float32)
            idx1 = jnp.zeros((_L,), jnp.int32)
            idx2 = jnp.zeros((_L,), jnp.int32)
            for e in range(E):
                v = s_v[e, pl.ds(g * _L, _L)]
                gt1 = v > best1
                gt2 = v > best2
                best2 = jnp.where(gt1, best1, jnp.where(gt2, v, best2))
                idx2 = jnp.where(gt1, idx1, jnp.where(gt2, e, idx2))
                best1 = jnp.where(gt1, v, best1)
                idx1 = jnp.where(gt1, e, idx1)
            for e in range(E):
                w_v[e, pl.ds(g * _L, _L)] = (
                    jnp.where(idx1 == e, best1, 0.0)
                    + jnp.where(idx2 == e, best2, 0.0))
    pltpu.sync_copy(wa_v, wa_hbm.at[:, pl.ds(base, _TOK_W)])
    pltpu.sync_copy(wb_v, wb_hbm.at[:, pl.ds(base, _TOK_W)])


def _route_call(scores_aT, scores_bT):
    mesh = plsc.VectorSubcoreMesh(core_axis_name="c", subcore_axis_name="s",
                                  num_cores=_NC, num_subcores=_NS)
    f = pl.kernel(
        _route_body,
        out_type=(jax.ShapeDtypeStruct((E, B), jnp.float32),
                  jax.ShapeDtypeStruct((E, B), jnp.float32)),
        mesh=mesh,
        scratch_types=[pltpu.VMEM((E, _TOK_W), jnp.float32),
                       pltpu.VMEM((E, _TOK_W), jnp.float32),
                       pltpu.VMEM((E, _TOK_W), jnp.float32),
                       pltpu.VMEM((E, _TOK_W), jnp.float32)],
        compiler_params=pltpu.CompilerParams(needs_layout_passes=False),
    )
    return f(scores_aT, scores_bT)


def _experts_body(x_ref, w1_ref, b1_ref, w2_ref, b2_ref, wa_ref, wb_ref,
                  agg_a_ref, agg_b_ref):
    x = x_ref[...]
    wT = jnp.transpose(wa_ref[...], (1, 0)).astype(jnp.bfloat16)
    wbTl = jnp.transpose(wb_ref[...], (1, 0)).astype(jnp.bfloat16)
    agg_a = None
    for e in range(E):
        h1 = _silu_fast(jnp.dot(x, w1_ref[e],
                                preferred_element_type=jnp.float32)
                        .astype(jnp.bfloat16) + b1_ref[e])
        h2 = _silu_fast(jnp.dot(h1, w2_ref[e],
                                preferred_element_type=jnp.float32)
                        .astype(jnp.bfloat16) + b2_ref[e])
        wa_col = wT[:, e:e + 1]
        wb_col = wbTl[:, e:e + 1]
        if agg_a is None:
            agg_a = wa_col * h2
            agg_b = wb_col * h2
        else:
            agg_a += wa_col * h2
            agg_b += wb_col * h2
    agg_a_ref[...] = agg_a
    agg_b_ref[...] = agg_b


def _heads_body(agg_a_ref, agg_b_ref,
                aW1_ref, ab1_ref, aW2_ref, ab2_ref,
                bW1_ref, bb1_ref, bW2_ref, bb2_ref,
                alpha_ref, beta_ref):
    for agg_ref, hW1_ref, hb1_ref, hW2_ref, hb2_ref, out_ref in (
            (agg_a_ref, aW1_ref, ab1_ref, aW2_ref, ab2_ref, alpha_ref),
            (agg_b_ref, bW1_ref, bb1_ref, bW2_ref, bb2_ref, beta_ref)):
        hh = _silu_fast(jnp.dot(agg_ref[...], hW1_ref[...],
                                preferred_element_type=jnp.float32)
                        + hb1_ref[...])
        logit = (jnp.sum(hh * hW2_ref[...], axis=1, keepdims=True)
                 + hb2_ref[...])
        out_ref[...] = jax.nn.softplus(logit)


def _heads_call(agg_a, agg_b, aW1b, ab1r, aW2r, ab2r, bW1b, bb1r, bW2r, bb2r):
    full = lambda shape: pl.BlockSpec(shape, lambda i: (0,) * len(shape))
    return pl.pallas_call(
        _heads_body,
        grid=(B // EXP_BT,),
        in_specs=[
            pl.BlockSpec((EXP_BT, EH), lambda i: (i, 0)),
            pl.BlockSpec((EXP_BT, EH), lambda i: (i, 0)),
            full((EH, EH)), full((1, EH)), full((1, EH)), full((1, 1)),
            full((EH, EH)), full((1, EH)), full((1, EH)), full((1, 1)),
        ],
        out_specs=[
            pl.BlockSpec((EXP_BT, 1), lambda i: (i, 0)),
            pl.BlockSpec((EXP_BT, 1), lambda i: (i, 0)),
        ],
        out_shape=[
            jax.ShapeDtypeStruct((B, 1), jnp.float32),
            jax.ShapeDtypeStruct((B, 1), jnp.float32),
        ],
        compiler_params=pltpu.CompilerParams(
            dimension_semantics=("arbitrary",)),
    )(agg_a, agg_b, aW1b, ab1r, aW2r, ab2r, bW1b, bb1r, bW2r, bb2r)


def _experts_call(xb, eW1b, eb1r, eW2b, eb2r, wa, wb):
    full = lambda shape: pl.BlockSpec(shape, lambda i: (0,) * len(shape))
    return pl.pallas_call(
        _experts_body,
        grid=(B // EXP_BT,),
        in_specs=[
            pl.BlockSpec((EXP_BT, CD), lambda i: (i, 0)),
            full((E, CD, EH)), full((E, 1, EH)),
            full((E, EH, EH)), full((E, 1, EH)),
            pl.BlockSpec((E, EXP_BT), lambda i: (0, i)),
            pl.BlockSpec((E, EXP_BT), lambda i: (0, i)),
        ],
        out_specs=[
            pl.BlockSpec((EXP_BT, EH), lambda i: (i, 0)),
            pl.BlockSpec((EXP_BT, EH), lambda i: (i, 0)),
        ],
        out_shape=[
            jax.ShapeDtypeStruct((B, EH), jnp.bfloat16),
            jax.ShapeDtypeStruct((B, EH), jnp.bfloat16),
        ],
        compiler_params=pltpu.CompilerParams(
            dimension_semantics=("arbitrary",),
            vmem_limit_bytes=100 * 1024 * 1024),
    )(xb, eW1b, eb1r, eW2b, eb2r, wa, wb)


def kernel(inputs, reference_base_embedding,
           gaW1, gab1, gaW2, gab2,
           gbW1, gbb1, gbW2, gbb2,
           eW1, eb1, eW2, eb2,
           aW1, ab1, aW2, ab2,
           bW1, bb1, bW2, bb2):
    (scores_a, scores_b, saT, sbT, xbf, eW1b, eW2b, aW1b, bW1b) = _gates_call(
        inputs, reference_base_embedding,
        gaW1, gab1.reshape(1, H), gaW2, gab2.reshape(1, E),
        gbW1, gbb1.reshape(1, H), gbW2, gbb2.reshape(1, E),
        eW1, eW2, aW1, bW1)

    waT, wbT = _route_call(saT, sbT)

    bf = jnp.bfloat16
    agg_a, agg_b = _experts_call(
        xbf,
        eW1b, eb1.reshape(E, 1, EH).astype(bf),
        eW2b, eb2.reshape(E, 1, EH).astype(bf),
        waT, wbT)

    alpha, beta = _heads_call(
        agg_a, agg_b,
        aW1b, ab1.reshape(1, EH), aW2.reshape(1, EH), ab2.reshape(1, 1),
        bW1b, bb1.reshape(1, EH), bW2.reshape(1, EH), bb2.reshape(1, 1))

    return (alpha, beta, scores_a, scores_b)
